# baseline (speedup 1.0000x reference)
import functools

import jax
import jax.numpy as jnp
from jax import lax
from jax.experimental import pallas as pl
from jax.experimental.pallas import tpu as pltpu
from jax.experimental.pallas import tpu_sc as plsc

N = 10000
E = 320000
D = 128
L = 16
NC = 2
NS = 16
NW = NC * NS
C = 128
CH = 80
EP = NW * CH * C
ER = EP // C
CHD = 2 * CH
NPAD = 10112
RW = NPAD // NS
SCH = 16
SCH2 = 40

_MESH = plsc.VectorSubcoreMesh(
    core_axis_name="c", subcore_axis_name="s", num_cores=NC, num_subcores=NS)
_SC_PARAMS = pltpu.CompilerParams(needs_layout_passes=False)


def _rsqrt16(d):
    pos = d > 0.0
    dsafe = jnp.where(pos, d, 1.0)
    i = lax.bitcast_convert_type(dsafe, jnp.int32)
    y = lax.bitcast_convert_type(jnp.int32(0x5F3759DF) - (i >> 1), jnp.float32)
    for _ in range(3):
        y = y * (1.5 - 0.5 * dsafe * y * y)
    return jnp.where(pos, y, 0.0)


def _scale_chunk(rows, norm_v, t):
    tt = jnp.full((L,), t, jnp.int32)

    def scale(i, _):
        ii = jnp.full((L,), i, jnp.int32)
        nrm = plsc.load_gather(norm_v, [tt, ii])
        for k in range(D // L):
            sl = pl.ds(k * L, L)
            rows[i, sl] = rows[i, sl] * nrm
        return 0

    lax.fori_loop(0, C, scale, 0, unroll=4)


def _prop_pass(table_hbm, idx_v, dst_v, norm_v, rows, gsem, ssem, acc_s,
               nch):
    pltpu.async_copy(table_hbm.at[idx_v.at[0]], rows[0], gsem[0])

    def pair(q, _):
        for b in range(2):
            t = q * 2 + b
            o = 1 - b

            @pl.when(t >= 1)
            def _drain():
                pltpu.make_async_copy(rows[o], acc_s.at[dst_v.at[t]],
                                      ssem[o]).wait()

            @pl.when(t + 1 < nch)
            def _prefetch():
                pltpu.async_copy(table_hbm.at[idx_v.at[t + 1]], rows[o],
                                 gsem[o])

            pltpu.make_async_copy(table_hbm.at[idx_v.at[t]], rows[b],
                                  gsem[b]).wait()
            _scale_chunk(rows[b], norm_v, t)
            pltpu.async_copy(rows[b], acc_s.at[dst_v.at[t]], ssem[b],
                             add=True)
        return 0

    lax.fori_loop(0, nch // 2, pair, 0)
    pltpu.make_async_copy(rows[1], acc_s.at[dst_v.at[nch - 1]],
                          ssem[1]).wait()


def _zero_acc_rows(zrow, rows0, acc_s, base):
    pltpu.sync_copy(zrow, rows0)
    for t in range(RW // C):
        pltpu.sync_copy(rows0, acc_s.at[pl.ds(base + t * C, C)])
    rem = RW % C
    if rem:
        pltpu.sync_copy(rows0.at[pl.ds(0, rem)],
                        acc_s.at[pl.ds(base + (RW // C) * C, rem)])


def _write_acc_rows(acc_s, rows0, out_hbm, c, base):
    nfull = RW // C
    for t in range(nfull):
        pltpu.sync_copy(acc_s.at[pl.ds(base + t * C, C)], rows0)
        pltpu.sync_copy(rows0, out_hbm.at[c, pl.ds(base + t * C, C)])
    rem = RW % C
    if rem:
        sl_s = pl.ds(base + nfull * C, rem)
        pltpu.sync_copy(acc_s.at[sl_s], rows0.at[pl.ds(0, rem)])
        pltpu.sync_copy(rows0.at[pl.ds(0, rem)], out_hbm.at[c, sl_s])


@functools.partial(
    pl.kernel,
    out_type=(jax.ShapeDtypeStruct((NC, NPAD, D), jnp.float32),
              jax.ShapeDtypeStruct((ER, C), jnp.float32)),
    mesh=_MESH,
    compiler_params=_SC_PARAMS,
    scratch_types=[
        pltpu.VMEM((SCH, C), jnp.int32),
        pltpu.VMEM((SCH, C), jnp.int32),
        pltpu.VMEM((SCH, C), jnp.float32),
        pltpu.VMEM((C, D), jnp.float32),
        pltpu.VMEM((C, D), jnp.float32),
        pltpu.VMEM((NPAD,), jnp.float32),
        pltpu.VMEM_SHARED((NPAD,), jnp.float32),
        pltpu.VMEM_SHARED((NPAD, D), jnp.float32),
        pltpu.SemaphoreType.DMA,
        pltpu.SemaphoreType.DMA,
        pltpu.SemaphoreType.DMA,
        pltpu.SemaphoreType.DMA,
        pltpu.SemaphoreType.DMA,
    ],
)
def _sc_stage1(x_hbm, srcf, dstf, attrf, zrow, zdeg, p1, normf,
               idx_v, dst_v, norm_v, rows0, rows1, dis_v, deg_s, acc_s,
               gsem0, gsem1, ssem0, ssem1, dsem):
    c = lax.axis_index("c")
    s = lax.axis_index("s")
    wrow = c * NS + s
    rows = (rows0, rows1)
    gsem = (gsem0, gsem1)
    ssem = (ssem0, ssem1)

    pltpu.sync_copy(zdeg, dis_v)
    pltpu.sync_copy(dis_v.at[pl.ds(s * RW, RW)], deg_s.at[pl.ds(s * RW, RW)])
    plsc.subcore_barrier()

    for p in range(CHD // SCH):
        base = s * CHD + p * SCH
        pltpu.sync_copy(srcf.at[pl.ds(base, SCH)], idx_v)
        pltpu.sync_copy(dstf.at[pl.ds(base, SCH)], dst_v)
        pltpu.sync_copy(attrf.at[pl.ds(base, SCH)], norm_v)

        def deg_chunk(j, _):
            for m in range(C // L):
                sl = pl.ds(m * L, L)
                eq = idx_v[j, sl] == dst_v[j, sl]
                norm_v[j, sl] = jnp.where(eq, 0.0, norm_v[j, sl])
            pltpu.async_copy(norm_v.at[j], deg_s.at[idx_v.at[j]], dsem,
                             add=True)
            return 0

        lax.fori_loop(0, SCH, deg_chunk, 0)

        def deg_drain(j, _):
            pltpu.make_async_copy(norm_v.at[j], deg_s.at[idx_v.at[j]],
                                  dsem).wait()
            return 0

        lax.fori_loop(0, SCH, deg_drain, 0)
    plsc.subcore_barrier()

    pltpu.sync_copy(deg_s, dis_v)

    def dis_blk(k, _):
        sl = pl.ds(k * L, L)
        dis_v[sl] = _rsqrt16(dis_v[sl])
        return 0

    lax.fori_loop(0, NPAD // L, dis_blk, 0, unroll=4)

    _zero_acc_rows(zrow, rows0, acc_s, s * RW)
    plsc.subcore_barrier()

    for p in range(CH // SCH):
        base = wrow * CH + p * SCH
        pltpu.sync_copy(srcf.at[pl.ds(base, SCH)], idx_v)
        pltpu.sync_copy(dstf.at[pl.ds(base, SCH)], dst_v)
        pltpu.sync_copy(attrf.at[pl.ds(base, SCH)], norm_v)

        def norm_blk(t, _):
            j = t // (C // L)
            m = t % (C // L)
            sl = pl.ds(m * L, L)
            s16 = idx_v[j, sl]
            d16 = dst_v[j, sl]
            w16 = jnp.where(s16 == d16, 0.0, norm_v[j, sl])
            ds_ = plsc.load_gather(dis_v, [s16])
            dd_ = plsc.load_gather(dis_v, [d16])
            norm_v[j, sl] = -(ds_ * w16) * dd_
            return 0

        lax.fori_loop(0, SCH * (C // L), norm_blk, 0, unroll=2)
        pltpu.sync_copy(norm_v, normf.at[pl.ds(base, SCH)])
        _prop_pass(x_hbm, idx_v, dst_v, norm_v, rows, gsem, ssem, acc_s,
                   SCH)

    plsc.subcore_barrier()
    _write_acc_rows(acc_s, rows0, p1, c, s * RW)


@functools.partial(
    pl.kernel,
    out_type=jax.ShapeDtypeStruct((NC, NPAD, D), jnp.float32),
    mesh=_MESH,
    compiler_params=_SC_PARAMS,
    scratch_types=[
        pltpu.VMEM((SCH2, C), jnp.int32),
        pltpu.VMEM((SCH2, C), jnp.int32),
        pltpu.VMEM((SCH2, C), jnp.float32),
        pltpu.VMEM((C, D), jnp.float32),
        pltpu.VMEM((C, D), jnp.float32),
        pltpu.VMEM_SHARED((NPAD, D), jnp.float32),
        pltpu.SemaphoreType.DMA,
        pltpu.SemaphoreType.DMA,
        pltpu.SemaphoreType.DMA,
        pltpu.SemaphoreType.DMA,
    ],
)
def _sc_stage2(tx1_hbm, srcf, dstf, normf, zrow, p2,
               idx_v, dst_v, norm_v, rows0, rows1, acc_s,
               gsem0, gsem1, ssem0, ssem1):
    c = lax.axis_index("c")
    s = lax.axis_index("s")
    wrow = c * NS + s
    rows = (rows0, rows1)
    gsem = (gsem0, gsem1)
    ssem = (ssem0, ssem1)

    _zero_acc_rows(zrow, rows0, acc_s, s * RW)
    plsc.subcore_barrier()

    for p in range(CH // SCH2):
        base = wrow * CH + p * SCH2
        pltpu.sync_copy(srcf.at[pl.ds(base, SCH2)], idx_v)
        pltpu.sync_copy(dstf.at[pl.ds(base, SCH2)], dst_v)
        pltpu.sync_copy(normf.at[pl.ds(base, SCH2)], norm_v)
        _prop_pass(tx1_hbm, idx_v, dst_v, norm_v, rows, gsem, ssem, acc_s,
                   SCH2)

    plsc.subcore_barrier()
    _write_acc_rows(acc_s, rows0, p2, c, s * RW)


def _sum2_body(p_ref, o_ref):
    o_ref[...] = p_ref[0] + p_ref[1]


def _sum_partials(p1):
    nb = 8
    return pl.pallas_call(
        _sum2_body,
        grid=(nb,),
        in_specs=[pl.BlockSpec((NC, NPAD // nb, D), lambda i: (0, i, 0))],
        out_specs=pl.BlockSpec((NPAD // nb, D), lambda i: (i, 0)),
        out_shape=jax.ShapeDtypeStruct((NPAD, D), jnp.float32),
    )(p1)


def _gates_body(x_ref, t1_ref, p2_ref, wz_ref, wh_ref, bz1_ref, bz2_ref,
                bh1_ref, bh2_ref, wl_ref, bl_ref, o_ref):
    xb = x_ref[...]
    t1 = t1_ref[...]
    t2 = 2.0 * (p2_ref[0] + p2_ref[1]) - xb
    f32 = jnp.float32
    z = (jnp.dot(xb, wz_ref[0], preferred_element_type=f32)
         + jnp.dot(t1, wz_ref[1], preferred_element_type=f32)
         + jnp.dot(t2, wz_ref[2], preferred_element_type=f32)
         + bz1_ref[...] + bz2_ref[...])
    h = (jnp.dot(xb, wh_ref[0], preferred_element_type=f32)
         + jnp.dot(t1, wh_ref[1], preferred_element_type=f32)
         + jnp.dot(t2, wh_ref[2], preferred_element_type=f32)
         + bh1_ref[...] + bh2_ref[...])
    zz = jax.nn.sigmoid(z)
    hh = jnp.tanh((1.0 - zz) * jnp.tanh(h))
    o_ref[...] = jax.nn.sigmoid(
        jnp.dot(hh, wl_ref[...], preferred_element_type=f32) + bl_ref[...])


def _gates(x, tx1, p2, W_xz, W_xh, b_xz, b_hz, b_xh, b_hh, W_lin, b_lin):
    bn = 1000
    nb = N // bn
    full = lambda *shape: pl.BlockSpec(shape, lambda i: (0,) * len(shape))
    return pl.pallas_call(
        _gates_body,
        grid=(nb,),
        in_specs=[
            pl.BlockSpec((bn, D), lambda i: (i, 0)),
            pl.BlockSpec((bn, D), lambda i: (i, 0)),
            pl.BlockSpec((NC, bn, D), lambda i: (0, i, 0)),
            full(3, D, D), full(3, D, D),
            full(1, D), full(1, D), full(1, D), full(1, D),
            full(D, 2), full(1, 2),
        ],
        out_specs=pl.BlockSpec((bn, 2), lambda i: (i, 0)),
        out_shape=jax.ShapeDtypeStruct((N, 2), jnp.float32),
    )(x, tx1, p2, W_xz, W_xh, b_xz.reshape(1, D), b_hz.reshape(1, D),
      b_xh.reshape(1, D), b_hh.reshape(1, D), W_lin, b_lin.reshape(1, 2))


def kernel(x, edge_index, edge_attr, W_xz, b_xz, W_hz, b_hz, W_xr, b_xr,
           W_hr, b_hr, W_xh, b_xh, W_hh, b_hh, W_lin, b_lin):
    pad = EP - E
    ar = jnp.arange(pad, dtype=jnp.int32)
    srcf = jnp.concatenate([edge_index[0], ar % N]).reshape(ER, C)
    dstf = jnp.concatenate([edge_index[1], N + ar % (NPAD - N)]).reshape(ER, C)
    attrf = jnp.pad(edge_attr, (0, pad)).reshape(ER, C)
    zrow = jnp.zeros((C, D), jnp.float32)
    zdeg = jnp.zeros((NPAD,), jnp.float32)

    p1, normf = _sc_stage1(x, srcf, dstf, attrf, zrow, zdeg)
    tx1 = _sum_partials(p1)
    p2 = _sc_stage2(tx1, srcf, dstf, normf, zrow)
    return _gates(x, tx1, p2, W_xz, W_xh, b_xz, b_hz, b_xh, b_hh,
                  W_lin, b_lin)


# speedup vs baseline: 1.0104x; 1.0104x over previous
import functools

import jax
import jax.numpy as jnp
from jax import lax
from jax.experimental import pallas as pl
from jax.experimental.pallas import tpu as pltpu
from jax.experimental.pallas import tpu_sc as plsc

N = 10000
E = 320000
D = 128
L = 16
NC = 2
NS = 16
NW = NC * NS
C = 128
CH = 80
EP = NW * CH * C
ER = EP // C
CHD = 2 * CH
NPAD = 10112
RW = NPAD // NS
SCH = 16
SCH2 = 40

_MESH = plsc.VectorSubcoreMesh(
    core_axis_name="c", subcore_axis_name="s", num_cores=NC, num_subcores=NS)
_SC_PARAMS = pltpu.CompilerParams(needs_layout_passes=False)


def _rsqrt16(d):
    pos = d > 0.0
    dsafe = jnp.where(pos, d, 1.0)
    i = lax.bitcast_convert_type(dsafe, jnp.int32)
    y = lax.bitcast_convert_type(jnp.int32(0x5F3759DF) - (i >> 1), jnp.float32)
    for _ in range(3):
        y = y * (1.5 - 0.5 * dsafe * y * y)
    return jnp.where(pos, y, 0.0)


def _scale_chunk(rows, norm_v, t):
    tt = jnp.full((L,), t, jnp.int32)

    def scale(i, _):
        ii = jnp.full((L,), i, jnp.int32)
        nrm = plsc.load_gather(norm_v, [tt, ii])
        for k in range(D // L):
            sl = pl.ds(k * L, L)
            rows[i, sl] = rows[i, sl] * nrm
        return 0

    lax.fori_loop(0, C, scale, 0, unroll=4)


def _prop_pass(table_hbm, idx_v, dst_v, norm_v, rows, gsem, ssem, acc_s,
               nch):
    pltpu.async_copy(table_hbm.at[idx_v.at[0]], rows[0], gsem[0])

    def pair(q, _):
        for b in range(2):
            t = q * 2 + b
            o = 1 - b

            @pl.when(t >= 1)
            def _drain():
                pltpu.make_async_copy(rows[o], acc_s.at[dst_v.at[t]],
                                      ssem[o]).wait()

            @pl.when(t + 1 < nch)
            def _prefetch():
                pltpu.async_copy(table_hbm.at[idx_v.at[t + 1]], rows[o],
                                 gsem[o])

            pltpu.make_async_copy(table_hbm.at[idx_v.at[t]], rows[b],
                                  gsem[b]).wait()
            _scale_chunk(rows[b], norm_v, t)
            pltpu.async_copy(rows[b], acc_s.at[dst_v.at[t]], ssem[b],
                             add=True)
        return 0

    lax.fori_loop(0, nch // 2, pair, 0)
    pltpu.make_async_copy(rows[1], acc_s.at[dst_v.at[nch - 1]],
                          ssem[1]).wait()


_WBLK = [(t * C, C) for t in range(RW // C)] + (
    [((RW // C) * C, RW % C)] if RW % C else [])


def _zero_acc_rows(zrow, rows0, sem, acc_s, base):
    pltpu.sync_copy(zrow, rows0)
    for off, sz in _WBLK:
        pltpu.async_copy(rows0.at[pl.ds(0, sz)],
                         acc_s.at[pl.ds(base + off, sz)], sem)
    for off, sz in _WBLK:
        pltpu.make_async_copy(rows0.at[pl.ds(0, sz)],
                              acc_s.at[pl.ds(base + off, sz)], sem).wait()


def _write_acc_rows(acc_s, rows, gsem, out_hbm, c, base):
    off0, sz0 = _WBLK[0]
    pltpu.async_copy(acc_s.at[pl.ds(base + off0, sz0)],
                     rows[0].at[pl.ds(0, sz0)], gsem[0])
    for t, (off, sz) in enumerate(_WBLK):
        b = t % 2
        pltpu.make_async_copy(acc_s.at[pl.ds(base + off, sz)],
                              rows[b].at[pl.ds(0, sz)], gsem[b]).wait()
        if t + 1 < len(_WBLK):
            noff, nsz = _WBLK[t + 1]
            pltpu.async_copy(acc_s.at[pl.ds(base + noff, nsz)],
                             rows[1 - b].at[pl.ds(0, nsz)], gsem[1 - b])
        pltpu.sync_copy(rows[b].at[pl.ds(0, sz)],
                        out_hbm.at[c, pl.ds(base + off, sz)])


@functools.partial(
    pl.kernel,
    out_type=(jax.ShapeDtypeStruct((NC, NPAD, D), jnp.float32),
              jax.ShapeDtypeStruct((ER, C), jnp.float32)),
    mesh=_MESH,
    compiler_params=_SC_PARAMS,
    scratch_types=[
        pltpu.VMEM((SCH, C), jnp.int32),
        pltpu.VMEM((SCH, C), jnp.int32),
        pltpu.VMEM((SCH, C), jnp.float32),
        pltpu.VMEM((C, D), jnp.float32),
        pltpu.VMEM((C, D), jnp.float32),
        pltpu.VMEM((NPAD,), jnp.float32),
        pltpu.VMEM_SHARED((NPAD,), jnp.float32),
        pltpu.VMEM_SHARED((NPAD, D), jnp.float32),
        pltpu.SemaphoreType.DMA,
        pltpu.SemaphoreType.DMA,
        pltpu.SemaphoreType.DMA,
        pltpu.SemaphoreType.DMA,
        pltpu.SemaphoreType.DMA,
    ],
)
def _sc_stage1(x_hbm, srcf, dstf, attrf, zrow, zdeg, p1, normf,
               idx_v, dst_v, norm_v, rows0, rows1, dis_v, deg_s, acc_s,
               gsem0, gsem1, ssem0, ssem1, dsem):
    c = lax.axis_index("c")
    s = lax.axis_index("s")
    wrow = c * NS + s
    rows = (rows0, rows1)
    gsem = (gsem0, gsem1)
    ssem = (ssem0, ssem1)

    pltpu.sync_copy(zdeg, dis_v)
    pltpu.sync_copy(dis_v.at[pl.ds(s * RW, RW)], deg_s.at[pl.ds(s * RW, RW)])
    plsc.subcore_barrier()

    for p in range(CHD // SCH):
        base = s * CHD + p * SCH
        pltpu.sync_copy(srcf.at[pl.ds(base, SCH)], idx_v)
        pltpu.sync_copy(dstf.at[pl.ds(base, SCH)], dst_v)
        pltpu.sync_copy(attrf.at[pl.ds(base, SCH)], norm_v)

        def deg_chunk(j, _):
            for m in range(C // L):
                sl = pl.ds(m * L, L)
                eq = idx_v[j, sl] == dst_v[j, sl]
                norm_v[j, sl] = jnp.where(eq, 0.0, norm_v[j, sl])
            pltpu.async_copy(norm_v.at[j], deg_s.at[idx_v.at[j]], dsem,
                             add=True)
            return 0

        lax.fori_loop(0, SCH, deg_chunk, 0)

        def deg_drain(j, _):
            pltpu.make_async_copy(norm_v.at[j], deg_s.at[idx_v.at[j]],
                                  dsem).wait()
            return 0

        lax.fori_loop(0, SCH, deg_drain, 0)
    plsc.subcore_barrier()

    pltpu.sync_copy(deg_s, dis_v)

    def dis_blk(k, _):
        sl = pl.ds(k * L, L)
        dis_v[sl] = _rsqrt16(dis_v[sl])
        return 0

    lax.fori_loop(0, NPAD // L, dis_blk, 0, unroll=4)

    _zero_acc_rows(zrow, rows0, ssem[0], acc_s, s * RW)
    plsc.subcore_barrier()

    for p in range(CH // SCH):
        base = wrow * CH + p * SCH
        pltpu.sync_copy(srcf.at[pl.ds(base, SCH)], idx_v)
        pltpu.sync_copy(dstf.at[pl.ds(base, SCH)], dst_v)
        pltpu.sync_copy(attrf.at[pl.ds(base, SCH)], norm_v)

        def norm_blk(t, _):
            j = t // (C // L)
            m = t % (C // L)
            sl = pl.ds(m * L, L)
            s16 = idx_v[j, sl]
            d16 = dst_v[j, sl]
            w16 = jnp.where(s16 == d16, 0.0, norm_v[j, sl])
            ds_ = plsc.load_gather(dis_v, [s16])
            dd_ = plsc.load_gather(dis_v, [d16])
            norm_v[j, sl] = -(ds_ * w16) * dd_
            return 0

        lax.fori_loop(0, SCH * (C // L), norm_blk, 0, unroll=2)
        pltpu.sync_copy(norm_v, normf.at[pl.ds(base, SCH)])
        _prop_pass(x_hbm, idx_v, dst_v, norm_v, rows, gsem, ssem, acc_s,
                   SCH)

    plsc.subcore_barrier()
    _write_acc_rows(acc_s, rows, gsem, p1, c, s * RW)


@functools.partial(
    pl.kernel,
    out_type=jax.ShapeDtypeStruct((NC, NPAD, D), jnp.float32),
    mesh=_MESH,
    compiler_params=_SC_PARAMS,
    scratch_types=[
        pltpu.VMEM((SCH2, C), jnp.int32),
        pltpu.VMEM((SCH2, C), jnp.int32),
        pltpu.VMEM((SCH2, C), jnp.float32),
        pltpu.VMEM((C, D), jnp.float32),
        pltpu.VMEM((C, D), jnp.float32),
        pltpu.VMEM_SHARED((NPAD, D), jnp.float32),
        pltpu.SemaphoreType.DMA,
        pltpu.SemaphoreType.DMA,
        pltpu.SemaphoreType.DMA,
        pltpu.SemaphoreType.DMA,
    ],
)
def _sc_stage2(tx1_hbm, srcf, dstf, normf, zrow, p2,
               idx_v, dst_v, norm_v, rows0, rows1, acc_s,
               gsem0, gsem1, ssem0, ssem1):
    c = lax.axis_index("c")
    s = lax.axis_index("s")
    wrow = c * NS + s
    rows = (rows0, rows1)
    gsem = (gsem0, gsem1)
    ssem = (ssem0, ssem1)

    _zero_acc_rows(zrow, rows0, ssem[0], acc_s, s * RW)
    plsc.subcore_barrier()

    for p in range(CH // SCH2):
        base = wrow * CH + p * SCH2
        pltpu.sync_copy(srcf.at[pl.ds(base, SCH2)], idx_v)
        pltpu.sync_copy(dstf.at[pl.ds(base, SCH2)], dst_v)
        pltpu.sync_copy(normf.at[pl.ds(base, SCH2)], norm_v)
        _prop_pass(tx1_hbm, idx_v, dst_v, norm_v, rows, gsem, ssem, acc_s,
                   SCH2)

    plsc.subcore_barrier()
    _write_acc_rows(acc_s, rows, gsem, p2, c, s * RW)


def _sum2_body(p_ref, o_ref):
    o_ref[...] = p_ref[0] + p_ref[1]


def _sum_partials(p1):
    nb = 8
    return pl.pallas_call(
        _sum2_body,
        grid=(nb,),
        in_specs=[pl.BlockSpec((NC, NPAD // nb, D), lambda i: (0, i, 0))],
        out_specs=pl.BlockSpec((NPAD // nb, D), lambda i: (i, 0)),
        out_shape=jax.ShapeDtypeStruct((NPAD, D), jnp.float32),
    )(p1)


def _gates_body(x_ref, t1_ref, p2_ref, wz_ref, wh_ref, bz1_ref, bz2_ref,
                bh1_ref, bh2_ref, wl_ref, bl_ref, o_ref):
    xb = x_ref[...]
    t1 = t1_ref[...]
    t2 = 2.0 * (p2_ref[0] + p2_ref[1]) - xb
    f32 = jnp.float32
    z = (jnp.dot(xb, wz_ref[0], preferred_element_type=f32)
         + jnp.dot(t1, wz_ref[1], preferred_element_type=f32)
         + jnp.dot(t2, wz_ref[2], preferred_element_type=f32)
         + bz1_ref[...] + bz2_ref[...])
    h = (jnp.dot(xb, wh_ref[0], preferred_element_type=f32)
         + jnp.dot(t1, wh_ref[1], preferred_element_type=f32)
         + jnp.dot(t2, wh_ref[2], preferred_element_type=f32)
         + bh1_ref[...] + bh2_ref[...])
    zz = jax.nn.sigmoid(z)
    hh = jnp.tanh((1.0 - zz) * jnp.tanh(h))
    o_ref[...] = jax.nn.sigmoid(
        jnp.dot(hh, wl_ref[...], preferred_element_type=f32) + bl_ref[...])


def _gates(x, tx1, p2, W_xz, W_xh, b_xz, b_hz, b_xh, b_hh, W_lin, b_lin):
    bn = 1000
    nb = N // bn
    full = lambda *shape: pl.BlockSpec(shape, lambda i: (0,) * len(shape))
    return pl.pallas_call(
        _gates_body,
        grid=(nb,),
        in_specs=[
            pl.BlockSpec((bn, D), lambda i: (i, 0)),
            pl.BlockSpec((bn, D), lambda i: (i, 0)),
            pl.BlockSpec((NC, bn, D), lambda i: (0, i, 0)),
            full(3, D, D), full(3, D, D),
            full(1, D), full(1, D), full(1, D), full(1, D),
            full(D, 2), full(1, 2),
        ],
        out_specs=pl.BlockSpec((bn, 2), lambda i: (i, 0)),
        out_shape=jax.ShapeDtypeStruct((N, 2), jnp.float32),
    )(x, tx1, p2, W_xz, W_xh, b_xz.reshape(1, D), b_hz.reshape(1, D),
      b_xh.reshape(1, D), b_hh.reshape(1, D), W_lin, b_lin.reshape(1, 2))


def kernel(x, edge_index, edge_attr, W_xz, b_xz, W_hz, b_hz, W_xr, b_xr,
           W_hr, b_hr, W_xh, b_xh, W_hh, b_hh, W_lin, b_lin):
    pad = EP - E
    ar = jnp.arange(pad, dtype=jnp.int32)
    srcf = jnp.concatenate([edge_index[0], ar % N]).reshape(ER, C)
    dstf = jnp.concatenate([edge_index[1], N + ar % (NPAD - N)]).reshape(ER, C)
    attrf = jnp.pad(edge_attr, (0, pad)).reshape(ER, C)
    zrow = jnp.zeros((C, D), jnp.float32)
    zdeg = jnp.zeros((NPAD,), jnp.float32)

    p1, normf = _sc_stage1(x, srcf, dstf, attrf, zrow, zdeg)
    tx1 = _sum_partials(p1)
    p2 = _sc_stage2(tx1, srcf, dstf, normf, zrow)
    return _gates(x, tx1, p2, W_xz, W_xh, b_xz, b_hz, b_xh, b_hh,
                  W_lin, b_lin)
